# baseline (speedup 1.0000x reference)
import functools

import jax
import jax.numpy as jnp
from jax import lax
from jax.experimental import pallas as pl
from jax.experimental.pallas import tpu as pltpu
from jax.experimental.pallas import tpu_sc as plsc

B = 16384
F = 26
V = 100000
E = 64
NNF = 13

BF = B * F
_NC, _NS = 2, 16
_NW = _NC * _NS
_ROWS_PER_W = BF // _NW
_CHUNK = 128
_NCHUNK = _ROWS_PER_W // _CHUNK
_NBUF = 4
_TOTAL_CHUNKS = BF // _CHUNK

_sc_mesh = plsc.VectorSubcoreMesh(
    core_axis_name="c", subcore_axis_name="s", num_cores=_NC, num_subcores=_NS
)


@functools.partial(
    pl.kernel,
    mesh=_sc_mesh,
    out_type=jax.ShapeDtypeStruct((BF, E), jnp.float32),
    scratch_types=[
        pltpu.VMEM((_NCHUNK, _CHUNK), jnp.int32),
        pltpu.VMEM((_CHUNK, E), jnp.float32),
        pltpu.VMEM((_CHUNK, E), jnp.float32),
        pltpu.VMEM((_CHUNK, E), jnp.float32),
        pltpu.VMEM((_CHUNK, E), jnp.float32),
        pltpu.SemaphoreType.DMA,
        pltpu.SemaphoreType.DMA,
        pltpu.SemaphoreType.DMA,
        pltpu.SemaphoreType.DMA,
    ],
    compiler_params=pltpu.CompilerParams(use_tc_tiling_on_sc=False),
)
def _sc_gather(table_hbm, idx_hbm, out_hbm, idx_v, b0, b1, b2, b3, s0, s1, s2, s3):
    wid = lax.axis_index("s") * _NC + lax.axis_index("c")
    chunk_base = wid * _NCHUNK
    row_base = wid * _ROWS_PER_W
    bufs = (b0, b1, b2, b3)
    sems = (s0, s1, s2, s3)

    pltpu.sync_copy(idx_hbm.at[pl.ds(chunk_base, _NCHUNK)], idx_v)

    for b in range(_NBUF):
        pltpu.make_async_copy(table_hbm.at[idx_v.at[b]], bufs[b], sems[b]).start()

    def body(g, carry):
        for b in range(_NBUF):
            j = g * _NBUF + b
            pltpu.make_async_copy(table_hbm.at[idx_v.at[j]], bufs[b], sems[b]).wait()
            pltpu.sync_copy(
                bufs[b], out_hbm.at[pl.ds(row_base + j * _CHUNK, _CHUNK)]
            )
            nj = j + _NBUF

            @pl.when(nj < _NCHUNK)
            def _():
                pltpu.make_async_copy(
                    table_hbm.at[idx_v.at[nj]], bufs[b], sems[b]
                ).start()

        return carry

    lax.fori_loop(0, _NCHUNK // _NBUF, body, 0)


_BB = 1024


def _mm_body(cat_ref, xn_ref, wcat_ref, wnum_ref, bnum_ref, wtail_ref, bfin_ref,
             out_ref):
    num = (
        jnp.dot(xn_ref[...], wnum_ref[...], preferred_element_type=jnp.float32)
        + bnum_ref[...]
    )
    acc = jnp.dot(cat_ref[...], wcat_ref[...], preferred_element_type=jnp.float32)
    acc = acc + jnp.dot(num, wtail_ref[...], preferred_element_type=jnp.float32)
    out_ref[...] = acc + bfin_ref[...]


_tc_matmul = pl.pallas_call(
    _mm_body,
    grid=(B // _BB,),
    in_specs=[
        pl.BlockSpec((_BB, F * E), lambda i: (i, 0)),
        pl.BlockSpec((_BB, NNF), lambda i: (i, 0)),
        pl.BlockSpec((F * E, E), lambda i: (0, 0)),
        pl.BlockSpec((NNF, E), lambda i: (0, 0)),
        pl.BlockSpec((1, E), lambda i: (0, 0)),
        pl.BlockSpec((E, E), lambda i: (0, 0)),
        pl.BlockSpec((1, E), lambda i: (0, 0)),
    ],
    out_specs=pl.BlockSpec((_BB, E), lambda i: (i, 0)),
    out_shape=jax.ShapeDtypeStruct((B, E), jnp.float32),
    compiler_params=pltpu.CompilerParams(
        dimension_semantics=("arbitrary",),
    ),
)


def kernel(X_cat, X_num, tables, W_num, b_num, W_final, b_final):
    table_flat = tables.reshape(F * V, E)
    idx = X_cat.astype(jnp.int32) + (jnp.arange(F, dtype=jnp.int32) * V)[None, :]
    idx = idx.reshape(_TOTAL_CHUNKS, _CHUNK)
    cat = _sc_gather(table_flat, idx)
    cat = cat.reshape(B, F * E)
    return _tc_matmul(
        cat,
        X_num,
        W_final[: F * E],
        W_num,
        b_num.reshape(1, E),
        W_final[F * E :],
        b_final.reshape(1, E),
    )


# speedup vs baseline: 1.5653x; 1.5653x over previous
import functools

import jax
import jax.numpy as jnp
from jax import lax
from jax.experimental import pallas as pl
from jax.experimental.pallas import tpu as pltpu
from jax.experimental.pallas import tpu_sc as plsc

B = 16384
F = 26
V = 100000
E = 64
NNF = 13

NPAIR = (7, 6)
PBASE = (0, 7)
VB = 2048
NVB = -(-V // VB)

_NC, _NS = 2, 16
_NW = _NC * _NS
_CHUNK = 128
_NBUF = 4


def _tr_body(a_ref, b_ref, out_ref):
    out_ref[0, :, :] = jnp.concatenate(
        [a_ref[0, :, :].T, b_ref[0, :, :].T], axis=1
    )


def _make_transpose(group):
    npair = NPAIR[group]
    base = PBASE[group]
    return pl.pallas_call(
        _tr_body,
        grid=(npair, NVB),
        in_specs=[
            pl.BlockSpec((1, E, VB), lambda g, j: (2 * (base + g), 0, j)),
            pl.BlockSpec((1, E, VB), lambda g, j: (2 * (base + g) + 1, 0, j)),
        ],
        out_specs=pl.BlockSpec((1, VB, 2 * E), lambda g, j: (g, j, 0)),
        out_shape=jax.ShapeDtypeStruct((npair, V, 2 * E), jnp.float32),
        compiler_params=pltpu.CompilerParams(
            dimension_semantics=("arbitrary", "arbitrary"),
        ),
    )


_transpose_calls = [_make_transpose(g) for g in range(2)]

_sc_mesh = plsc.VectorSubcoreMesh(
    core_axis_name="c", subcore_axis_name="s", num_cores=_NC, num_subcores=_NS
)


def _make_sc_gather(rows):
    nchunk = rows // (_NW * _CHUNK)
    rows_per_w = rows // _NW

    @functools.partial(
        pl.kernel,
        mesh=_sc_mesh,
        out_type=jax.ShapeDtypeStruct((rows, E), jnp.float32),
        scratch_types=[
            pltpu.VMEM((nchunk, _CHUNK), jnp.int32),
            pltpu.VMEM((_CHUNK, E), jnp.float32),
            pltpu.VMEM((_CHUNK, E), jnp.float32),
            pltpu.VMEM((_CHUNK, E), jnp.float32),
            pltpu.VMEM((_CHUNK, E), jnp.float32),
            pltpu.SemaphoreType.DMA,
            pltpu.SemaphoreType.DMA,
            pltpu.SemaphoreType.DMA,
            pltpu.SemaphoreType.DMA,
        ],
        compiler_params=pltpu.CompilerParams(use_tc_tiling_on_sc=False),
    )
    def _sc_gather(table_hbm, idx_hbm, out_hbm,
                   idx_v, b0, b1, b2, b3, s0, s1, s2, s3):
        wid = lax.axis_index("s") * _NC + lax.axis_index("c")
        chunk_base = wid * nchunk
        row_base = wid * rows_per_w
        bufs = (b0, b1, b2, b3)
        sems = (s0, s1, s2, s3)

        pltpu.sync_copy(idx_hbm.at[pl.ds(chunk_base, nchunk)], idx_v)

        for b in range(_NBUF):
            pltpu.make_async_copy(
                table_hbm.at[idx_v.at[b]], bufs[b], sems[b]
            ).start()

        def body(g, carry):
            for b in range(_NBUF):
                j = g * _NBUF + b
                pltpu.make_async_copy(
                    table_hbm.at[idx_v.at[j]], bufs[b], sems[b]
                ).wait()
                pltpu.sync_copy(
                    bufs[b], out_hbm.at[pl.ds(row_base + j * _CHUNK, _CHUNK)]
                )
                nj = j + _NBUF

                @pl.when(nj < nchunk)
                def _():
                    pltpu.make_async_copy(
                        table_hbm.at[idx_v.at[nj]], bufs[b], sems[b]
                    ).start()

            return carry

        lax.fori_loop(0, nchunk // _NBUF, body, 0)

    return _sc_gather


_sc_gathers = [_make_sc_gather(B * 2 * NPAIR[g]) for g in range(2)]

_BB = 1024
_C0 = 2 * NPAIR[0] * E
_C1 = 2 * NPAIR[1] * E


def _mm_body(g0_ref, g1_ref, xn_ref, w0_ref, w1_ref, wnum_ref, bnum_ref,
             wtail_ref, bfin_ref, out_ref):
    num = (
        jnp.dot(xn_ref[...], wnum_ref[...], preferred_element_type=jnp.float32)
        + bnum_ref[...]
    )
    acc = jnp.dot(g0_ref[...], w0_ref[...], preferred_element_type=jnp.float32)
    acc = acc + jnp.dot(g1_ref[...], w1_ref[...], preferred_element_type=jnp.float32)
    acc = acc + jnp.dot(num, wtail_ref[...], preferred_element_type=jnp.float32)
    out_ref[...] = acc + bfin_ref[...]


_tc_matmul = pl.pallas_call(
    _mm_body,
    grid=(B // _BB,),
    in_specs=[
        pl.BlockSpec((_BB, _C0), lambda i: (i, 0)),
        pl.BlockSpec((_BB, _C1), lambda i: (i, 0)),
        pl.BlockSpec((_BB, NNF), lambda i: (i, 0)),
        pl.BlockSpec((_C0, E), lambda i: (0, 0)),
        pl.BlockSpec((_C1, E), lambda i: (0, 0)),
        pl.BlockSpec((NNF, E), lambda i: (0, 0)),
        pl.BlockSpec((1, E), lambda i: (0, 0)),
        pl.BlockSpec((E, E), lambda i: (0, 0)),
        pl.BlockSpec((1, E), lambda i: (0, 0)),
    ],
    out_specs=pl.BlockSpec((_BB, E), lambda i: (i, 0)),
    out_shape=jax.ShapeDtypeStruct((B, E), jnp.float32),
    compiler_params=pltpu.CompilerParams(
        dimension_semantics=("arbitrary",),
    ),
)


def kernel(X_cat, X_num, tables, W_num, b_num, W_final, b_final):
    tt = jnp.transpose(tables, (0, 2, 1))
    xc = X_cat.astype(jnp.int32)
    gs = []
    col_base = 0
    for g in range(2):
        nf = 2 * NPAIR[g]
        packed = _transpose_calls[g](tt, tt)
        flat = packed.reshape(NPAIR[g] * V * 2, E)
        fk = jnp.arange(nf, dtype=jnp.int32)
        idx = 2 * ((fk // 2)[None, :] * V + xc[:, col_base:col_base + nf]) \
            + (fk % 2)[None, :]
        idx = idx.reshape(B * nf // _CHUNK, _CHUNK)
        gath = _sc_gathers[g](flat, idx)
        gs.append(gath.reshape(B, nf * E))
        col_base += nf
    return _tc_matmul(
        gs[0],
        gs[1],
        X_num,
        W_final[:_C0],
        W_final[_C0:_C0 + _C1],
        W_num,
        b_num.reshape(1, E),
        W_final[_C0 + _C1:],
        b_final.reshape(1, E),
    )
